# baseline (speedup 1.0000x reference)
import jax
import jax.numpy as jnp
from jax import lax
from jax.experimental import pallas as pl
from jax.experimental.pallas import tpu as pltpu
from jax.experimental.pallas import tpu_sc as plsc

CODEBOOK = 8192
DIM = 64
BETA = 0.25
NTOK = 8192

NT = 256
KT = 256
GW = 128


def _dist_argmin_kernel(z_ref, w_ref, idx_ref, mind_ref):
    zn = z_ref[...]
    xsq = jnp.sum(zn * zn, axis=1, keepdims=True)
    zm2 = -2.0 * zn

    def half_argmin(k0):
        def body(kc, carry):
            m_run, i_run = carry
            w = w_ref[pl.ds(k0 + kc * KT, KT), :]
            wsq = jnp.sum(w * w, axis=1)
            xw2 = lax.dot_general(zm2, w, (((1,), (1,)), ((), ())),
                                  preferred_element_type=jnp.float32)
            d = (xsq + wsq[None, :]) + xw2
            m = jnp.min(d, axis=1)
            kk = lax.broadcasted_iota(jnp.int32, (NT, KT), 1)
            li = jnp.min(jnp.where(d == m[:, None], kk, jnp.int32(2**30)),
                         axis=1) + (k0 + kc * KT)
            better = m < m_run
            return jnp.where(better, m, m_run), jnp.where(better, li, i_run)

        m0 = jnp.full((NT,), jnp.inf, jnp.float32)
        i0 = jnp.zeros((NT,), jnp.int32)
        return lax.fori_loop(0, (CODEBOOK // 2) // KT, body, (m0, i0))

    m1, i1 = half_argmin(0)
    m2, i2 = half_argmin(CODEBOOK // 2)
    m1b = m1.astype(jnp.bfloat16).astype(jnp.float32)
    take2 = m2 < m1b
    idx_ref[0, 0, :] = jnp.where(take2, i2, i1)
    mind_ref[0, 0, :] = jnp.where(take2, m2, m1)


def _dist_argmin(z_flat, weight):
    grid = (NTOK // NT,)
    return pl.pallas_call(
        _dist_argmin_kernel,
        grid=grid,
        in_specs=[
            pl.BlockSpec((NT, DIM), lambda i: (i, 0)),
            pl.BlockSpec((CODEBOOK, DIM), lambda i: (0, 0)),
        ],
        out_specs=(
            pl.BlockSpec((1, 1, NT), lambda i: (i, 0, 0)),
            pl.BlockSpec((1, 1, NT), lambda i: (i, 0, 0)),
        ),
        out_shape=(
            jax.ShapeDtypeStruct((NTOK // NT, 1, NT), jnp.int32),
            jax.ShapeDtypeStruct((NTOK // NT, 1, NT), jnp.float32),
        ),
    )(z_flat, weight)


def _sc_gather(weight, idx):
    w_pad = jnp.concatenate([weight, jnp.zeros_like(weight)], axis=1)
    idx2 = idx.reshape(1, NTOK)
    mesh = plsc.VectorSubcoreMesh(core_axis_name="core",
                                  subcore_axis_name="subcore")

    @pl.kernel(out_type=jax.ShapeDtypeStruct((NTOK, 2 * DIM), weight.dtype),
               mesh=mesh)
    def kern(w_hbm, i_hbm, o_hbm):
        def body(i_vmem, o_vmem):
            pltpu.sync_copy(w_hbm.at[i_vmem.at[0]], o_vmem)

        pltpu.emit_pipeline(
            body,
            grid=(NTOK // GW,),
            in_specs=[pl.BlockSpec((1, GW), lambda i: (0, i))],
            out_specs=[pl.BlockSpec((GW, 2 * DIM), lambda i: (i, 0))],
            core_axis_name="subcore",
            dimension_semantics=(pltpu.PARALLEL,),
        )(i_hbm, o_hbm)

    return kern(w_pad, idx2)[:, :DIM]


def kernel(z, weight):
    b, c, h, w = z.shape
    z_perm = jnp.transpose(z, (0, 2, 3, 1))
    nrm = jnp.linalg.norm(z_perm, axis=-1, keepdims=True)
    z_flat = (z_perm / jnp.clip(nrm, 1e-12, None)).reshape(NTOK, DIM)
    idx3, mind3 = _dist_argmin(z_flat, weight)
    idx = idx3.reshape(NTOK)
    loss = BETA * jnp.sum(mind3) / (NTOK * DIM)
    z_q = _sc_gather(weight, idx)
    z_q_out = jnp.transpose(z_q.reshape(b, h, w, c), (0, 3, 1, 2))
    return z_q_out, loss, idx


# speedup vs baseline: 2.3785x; 2.3785x over previous
import jax
import jax.numpy as jnp
from jax import lax
from jax.experimental import pallas as pl
from jax.experimental.pallas import tpu as pltpu
from jax.experimental.pallas import tpu_sc as plsc

CODEBOOK = 8192
DIM = 64
BETA = 0.25
NTOK = 8192

NT = 256
KT = 1024
GW = 128


def _dist_argmin_kernel(z_ref, w_ref, idx_ref, mind_ref):
    zn = z_ref[...]
    xsq = jnp.sum(zn * zn, axis=1, keepdims=True)
    zm2 = -2.0 * zn

    def half_argmin(k0):
        def body(kc, carry):
            m_run, i_run = carry
            w = w_ref[pl.ds(k0 + kc * KT, KT), :]
            wsq = jnp.sum(w * w, axis=1)
            xw2 = lax.dot_general(zm2, w, (((1,), (1,)), ((), ())),
                                  preferred_element_type=jnp.float32)
            d = (xsq + wsq[None, :]) + xw2
            m = jnp.min(d, axis=1)
            kk = lax.broadcasted_iota(jnp.int32, (NT, KT), 1)
            li = jnp.min(jnp.where(d == m[:, None], kk, jnp.int32(2**30)),
                         axis=1) + (k0 + kc * KT)
            better = m < m_run
            return jnp.where(better, m, m_run), jnp.where(better, li, i_run)

        m0 = jnp.full((NT,), jnp.inf, jnp.float32)
        i0 = jnp.zeros((NT,), jnp.int32)
        return lax.fori_loop(0, (CODEBOOK // 2) // KT, body, (m0, i0))

    m1, i1 = half_argmin(0)
    m2, i2 = half_argmin(CODEBOOK // 2)
    m1b = m1.astype(jnp.bfloat16).astype(jnp.float32)
    take2 = m2 < m1b
    idx_ref[0, 0, :] = jnp.where(take2, i2, i1)
    mind_ref[0, 0, :] = jnp.where(take2, m2, m1)


def _dist_argmin(z_flat, weight):
    grid = (NTOK // NT,)
    return pl.pallas_call(
        _dist_argmin_kernel,
        grid=grid,
        in_specs=[
            pl.BlockSpec((NT, DIM), lambda i: (i, 0)),
            pl.BlockSpec((CODEBOOK, DIM), lambda i: (0, 0)),
        ],
        out_specs=(
            pl.BlockSpec((1, 1, NT), lambda i: (i, 0, 0)),
            pl.BlockSpec((1, 1, NT), lambda i: (i, 0, 0)),
        ),
        out_shape=(
            jax.ShapeDtypeStruct((NTOK // NT, 1, NT), jnp.int32),
            jax.ShapeDtypeStruct((NTOK // NT, 1, NT), jnp.float32),
        ),
    )(z_flat, weight)


def _sc_gather(weight, idx):
    w_pad = jnp.concatenate([weight, jnp.zeros_like(weight)], axis=1)
    idx2 = idx.reshape(1, NTOK)
    mesh = plsc.VectorSubcoreMesh(core_axis_name="core",
                                  subcore_axis_name="subcore")

    @pl.kernel(out_type=jax.ShapeDtypeStruct((NTOK, 2 * DIM), weight.dtype),
               mesh=mesh)
    def kern(w_hbm, i_hbm, o_hbm):
        def body(i_vmem, o_vmem):
            pltpu.sync_copy(w_hbm.at[i_vmem.at[0]], o_vmem)

        pltpu.emit_pipeline(
            body,
            grid=(NTOK // GW,),
            in_specs=[pl.BlockSpec((1, GW), lambda i: (0, i))],
            out_specs=[pl.BlockSpec((GW, 2 * DIM), lambda i: (i, 0))],
            core_axis_name="subcore",
            dimension_semantics=(pltpu.PARALLEL,),
        )(i_hbm, o_hbm)

    return kern(w_pad, idx2)[:, :DIM]


def kernel(z, weight):
    b, c, h, w = z.shape
    z_perm = jnp.transpose(z, (0, 2, 3, 1))
    nrm = jnp.linalg.norm(z_perm, axis=-1, keepdims=True)
    z_flat = (z_perm / jnp.clip(nrm, 1e-12, None)).reshape(NTOK, DIM)
    idx3, mind3 = _dist_argmin(z_flat, weight)
    idx = idx3.reshape(NTOK)
    loss = BETA * jnp.sum(mind3) / (NTOK * DIM)
    z_q = _sc_gather(weight, idx)
    z_q_out = jnp.transpose(z_q.reshape(b, h, w, c), (0, 3, 1, 2))
    return z_q_out, loss, idx


# speedup vs baseline: 2.8780x; 1.2100x over previous
import jax
import jax.numpy as jnp
from jax import lax
from jax.experimental import pallas as pl
from jax.experimental.pallas import tpu as pltpu
from jax.experimental.pallas import tpu_sc as plsc

CODEBOOK = 8192
DIM = 64
BETA = 0.25
NTOK = 8192

NT = 256
KT = 2048
GW = 128


def _dist_argmin_kernel(z_ref, w_ref, idx_ref, mind_ref):
    zn = z_ref[...]
    xsq = jnp.sum(zn * zn, axis=1, keepdims=True)
    zm2 = -2.0 * zn

    def half_argmin(k0):
        def body(kc, carry):
            m_run, i_run = carry
            w = w_ref[pl.ds(k0 + kc * KT, KT), :]
            wsq = jnp.sum(w * w, axis=1)
            xw2 = lax.dot_general(zm2, w, (((1,), (1,)), ((), ())),
                                  preferred_element_type=jnp.float32)
            d = (xsq + wsq[None, :]) + xw2
            m = jnp.min(d, axis=1)
            kk = lax.broadcasted_iota(jnp.int32, (NT, KT), 1)
            li = jnp.min(jnp.where(d == m[:, None], kk, jnp.int32(2**30)),
                         axis=1) + (k0 + kc * KT)
            better = m < m_run
            return jnp.where(better, m, m_run), jnp.where(better, li, i_run)

        m0 = jnp.full((NT,), jnp.inf, jnp.float32)
        i0 = jnp.zeros((NT,), jnp.int32)
        return lax.fori_loop(0, (CODEBOOK // 2) // KT, body, (m0, i0))

    m1, i1 = half_argmin(0)
    m2, i2 = half_argmin(CODEBOOK // 2)
    m1b = m1.astype(jnp.bfloat16).astype(jnp.float32)
    take2 = m2 < m1b
    idx_ref[0, 0, :] = jnp.where(take2, i2, i1)
    mind_ref[0, 0, :] = jnp.where(take2, m2, m1)


def _dist_argmin(z_flat, weight):
    grid = (NTOK // NT,)
    return pl.pallas_call(
        _dist_argmin_kernel,
        grid=grid,
        in_specs=[
            pl.BlockSpec((NT, DIM), lambda i: (i, 0)),
            pl.BlockSpec((CODEBOOK, DIM), lambda i: (0, 0)),
        ],
        out_specs=(
            pl.BlockSpec((1, 1, NT), lambda i: (i, 0, 0)),
            pl.BlockSpec((1, 1, NT), lambda i: (i, 0, 0)),
        ),
        out_shape=(
            jax.ShapeDtypeStruct((NTOK // NT, 1, NT), jnp.int32),
            jax.ShapeDtypeStruct((NTOK // NT, 1, NT), jnp.float32),
        ),
    )(z_flat, weight)


def _sc_gather(weight, idx):
    w_pad = jnp.concatenate([weight, jnp.zeros_like(weight)], axis=1)
    idx2 = idx.reshape(1, NTOK)
    mesh = plsc.VectorSubcoreMesh(core_axis_name="core",
                                  subcore_axis_name="subcore")

    @pl.kernel(out_type=jax.ShapeDtypeStruct((NTOK, 2 * DIM), weight.dtype),
               mesh=mesh)
    def kern(w_hbm, i_hbm, o_hbm):
        def body(i_vmem, o_vmem):
            pltpu.sync_copy(w_hbm.at[i_vmem.at[0]], o_vmem)

        pltpu.emit_pipeline(
            body,
            grid=(NTOK // GW,),
            in_specs=[pl.BlockSpec((1, GW), lambda i: (0, i))],
            out_specs=[pl.BlockSpec((GW, 2 * DIM), lambda i: (i, 0))],
            core_axis_name="subcore",
            dimension_semantics=(pltpu.PARALLEL,),
        )(i_hbm, o_hbm)

    return kern(w_pad, idx2)[:, :DIM]


def kernel(z, weight):
    b, c, h, w = z.shape
    z_perm = jnp.transpose(z, (0, 2, 3, 1))
    nrm = jnp.linalg.norm(z_perm, axis=-1, keepdims=True)
    z_flat = (z_perm / jnp.clip(nrm, 1e-12, None)).reshape(NTOK, DIM)
    idx3, mind3 = _dist_argmin(z_flat, weight)
    idx = idx3.reshape(NTOK)
    loss = BETA * jnp.sum(mind3) / (NTOK * DIM)
    z_q = _sc_gather(weight, idx)
    z_q_out = jnp.transpose(z_q.reshape(b, h, w, c), (0, 3, 1, 2))
    return z_q_out, loss, idx


# speedup vs baseline: 3.2838x; 1.1410x over previous
import jax
import jax.numpy as jnp
from jax import lax
from jax.experimental import pallas as pl
from jax.experimental.pallas import tpu as pltpu
from jax.experimental.pallas import tpu_sc as plsc

CODEBOOK = 8192
DIM = 64
BETA = 0.25
NTOK = 8192

NT = 256
KT = 4096
GW = 128


def _dist_argmin_kernel(z_ref, w_ref, idx_ref, mind_ref):
    zn = z_ref[...]
    xsq = jnp.sum(zn * zn, axis=1, keepdims=True)
    zm2 = -2.0 * zn

    def half_argmin(k0):
        def body(kc, carry):
            m_run, i_run = carry
            w = w_ref[pl.ds(k0 + kc * KT, KT), :]
            wsq = jnp.sum(w * w, axis=1)
            xw2 = lax.dot_general(zm2, w, (((1,), (1,)), ((), ())),
                                  preferred_element_type=jnp.float32)
            d = (xsq + wsq[None, :]) + xw2
            m = jnp.min(d, axis=1)
            kk = lax.broadcasted_iota(jnp.int32, (NT, KT), 1)
            li = jnp.min(jnp.where(d == m[:, None], kk, jnp.int32(2**30)),
                         axis=1) + (k0 + kc * KT)
            better = m < m_run
            return jnp.where(better, m, m_run), jnp.where(better, li, i_run)

        m0 = jnp.full((NT,), jnp.inf, jnp.float32)
        i0 = jnp.zeros((NT,), jnp.int32)
        return lax.fori_loop(0, (CODEBOOK // 2) // KT, body, (m0, i0))

    m1, i1 = half_argmin(0)
    m2, i2 = half_argmin(CODEBOOK // 2)
    m1b = m1.astype(jnp.bfloat16).astype(jnp.float32)
    take2 = m2 < m1b
    idx_ref[0, 0, :] = jnp.where(take2, i2, i1)
    mind_ref[0, 0, :] = jnp.where(take2, m2, m1)


def _dist_argmin(z_flat, weight):
    grid = (NTOK // NT,)
    return pl.pallas_call(
        _dist_argmin_kernel,
        grid=grid,
        in_specs=[
            pl.BlockSpec((NT, DIM), lambda i: (i, 0)),
            pl.BlockSpec((CODEBOOK, DIM), lambda i: (0, 0)),
        ],
        out_specs=(
            pl.BlockSpec((1, 1, NT), lambda i: (i, 0, 0)),
            pl.BlockSpec((1, 1, NT), lambda i: (i, 0, 0)),
        ),
        out_shape=(
            jax.ShapeDtypeStruct((NTOK // NT, 1, NT), jnp.int32),
            jax.ShapeDtypeStruct((NTOK // NT, 1, NT), jnp.float32),
        ),
    )(z_flat, weight)


def _sc_gather(weight, idx):
    w_pad = jnp.concatenate([weight, jnp.zeros_like(weight)], axis=1)
    idx2 = idx.reshape(1, NTOK)
    mesh = plsc.VectorSubcoreMesh(core_axis_name="core",
                                  subcore_axis_name="subcore")

    @pl.kernel(out_type=jax.ShapeDtypeStruct((NTOK, 2 * DIM), weight.dtype),
               mesh=mesh)
    def kern(w_hbm, i_hbm, o_hbm):
        def body(i_vmem, o_vmem):
            pltpu.sync_copy(w_hbm.at[i_vmem.at[0]], o_vmem)

        pltpu.emit_pipeline(
            body,
            grid=(NTOK // GW,),
            in_specs=[pl.BlockSpec((1, GW), lambda i: (0, i))],
            out_specs=[pl.BlockSpec((GW, 2 * DIM), lambda i: (i, 0))],
            core_axis_name="subcore",
            dimension_semantics=(pltpu.PARALLEL,),
        )(i_hbm, o_hbm)

    return kern(w_pad, idx2)[:, :DIM]


def kernel(z, weight):
    b, c, h, w = z.shape
    z_perm = jnp.transpose(z, (0, 2, 3, 1))
    nrm = jnp.linalg.norm(z_perm, axis=-1, keepdims=True)
    z_flat = (z_perm / jnp.clip(nrm, 1e-12, None)).reshape(NTOK, DIM)
    idx3, mind3 = _dist_argmin(z_flat, weight)
    idx = idx3.reshape(NTOK)
    loss = BETA * jnp.sum(mind3) / (NTOK * DIM)
    z_q = _sc_gather(weight, idx)
    z_q_out = jnp.transpose(z_q.reshape(b, h, w, c), (0, 3, 1, 2))
    return z_q_out, loss, idx


# speedup vs baseline: 3.7150x; 1.1313x over previous
import jax
import jax.numpy as jnp
from jax import lax
from jax.experimental import pallas as pl
from jax.experimental.pallas import tpu as pltpu
from jax.experimental.pallas import tpu_sc as plsc

CODEBOOK = 8192
DIM = 64
BETA = 0.25
NTOK = 8192

NT = 512
KT = 4096
GW = 128


def _dist_argmin_kernel(z_ref, w_ref, idx_ref, mind_ref):
    zn = z_ref[...]
    xsq = jnp.sum(zn * zn, axis=1, keepdims=True)
    zm2 = -2.0 * zn

    def half_argmin(k0):
        def body(kc, carry):
            m_run, i_run = carry
            w = w_ref[pl.ds(k0 + kc * KT, KT), :]
            wsq = jnp.sum(w * w, axis=1)
            xw2 = lax.dot_general(zm2, w, (((1,), (1,)), ((), ())),
                                  preferred_element_type=jnp.float32)
            d = (xsq + wsq[None, :]) + xw2
            m = jnp.min(d, axis=1)
            kk = lax.broadcasted_iota(jnp.int32, (NT, KT), 1)
            li = jnp.min(jnp.where(d == m[:, None], kk, jnp.int32(2**30)),
                         axis=1) + (k0 + kc * KT)
            better = m < m_run
            return jnp.where(better, m, m_run), jnp.where(better, li, i_run)

        m0 = jnp.full((NT,), jnp.inf, jnp.float32)
        i0 = jnp.zeros((NT,), jnp.int32)
        return lax.fori_loop(0, (CODEBOOK // 2) // KT, body, (m0, i0))

    m1, i1 = half_argmin(0)
    m2, i2 = half_argmin(CODEBOOK // 2)
    m1b = m1.astype(jnp.bfloat16).astype(jnp.float32)
    take2 = m2 < m1b
    idx_ref[0, 0, :] = jnp.where(take2, i2, i1)
    mind_ref[0, 0, :] = jnp.where(take2, m2, m1)


def _dist_argmin(z_flat, weight):
    grid = (NTOK // NT,)
    return pl.pallas_call(
        _dist_argmin_kernel,
        grid=grid,
        in_specs=[
            pl.BlockSpec((NT, DIM), lambda i: (i, 0)),
            pl.BlockSpec((CODEBOOK, DIM), lambda i: (0, 0)),
        ],
        out_specs=(
            pl.BlockSpec((1, 1, NT), lambda i: (i, 0, 0)),
            pl.BlockSpec((1, 1, NT), lambda i: (i, 0, 0)),
        ),
        out_shape=(
            jax.ShapeDtypeStruct((NTOK // NT, 1, NT), jnp.int32),
            jax.ShapeDtypeStruct((NTOK // NT, 1, NT), jnp.float32),
        ),
    )(z_flat, weight)


def _sc_gather(weight, idx):
    w_pad = jnp.concatenate([weight, jnp.zeros_like(weight)], axis=1)
    idx2 = idx.reshape(1, NTOK)
    mesh = plsc.VectorSubcoreMesh(core_axis_name="core",
                                  subcore_axis_name="subcore")

    @pl.kernel(out_type=jax.ShapeDtypeStruct((NTOK, 2 * DIM), weight.dtype),
               mesh=mesh)
    def kern(w_hbm, i_hbm, o_hbm):
        def body(i_vmem, o_vmem):
            pltpu.sync_copy(w_hbm.at[i_vmem.at[0]], o_vmem)

        pltpu.emit_pipeline(
            body,
            grid=(NTOK // GW,),
            in_specs=[pl.BlockSpec((1, GW), lambda i: (0, i))],
            out_specs=[pl.BlockSpec((GW, 2 * DIM), lambda i: (i, 0))],
            core_axis_name="subcore",
            dimension_semantics=(pltpu.PARALLEL,),
        )(i_hbm, o_hbm)

    return kern(w_pad, idx2)[:, :DIM]


def kernel(z, weight):
    b, c, h, w = z.shape
    z_perm = jnp.transpose(z, (0, 2, 3, 1))
    nrm = jnp.linalg.norm(z_perm, axis=-1, keepdims=True)
    z_flat = (z_perm / jnp.clip(nrm, 1e-12, None)).reshape(NTOK, DIM)
    idx3, mind3 = _dist_argmin(z_flat, weight)
    idx = idx3.reshape(NTOK)
    loss = BETA * jnp.sum(mind3) / (NTOK * DIM)
    z_q = _sc_gather(weight, idx)
    z_q_out = jnp.transpose(z_q.reshape(b, h, w, c), (0, 3, 1, 2))
    return z_q_out, loss, idx


# speedup vs baseline: 3.7951x; 1.0216x over previous
import jax
import jax.numpy as jnp
from jax import lax
from jax.experimental import pallas as pl
from jax.experimental.pallas import tpu as pltpu
from jax.experimental.pallas import tpu_sc as plsc

CODEBOOK = 8192
DIM = 64
BETA = 0.25
NTOK = 8192

NT = 1024
KT = 4096
GW = 128


def _dist_argmin_kernel(z_ref, w_ref, idx_ref, mind_ref):
    zn = z_ref[...]
    xsq = jnp.sum(zn * zn, axis=1, keepdims=True)
    zm2 = -2.0 * zn

    def half_argmin(k0):
        def body(kc, carry):
            m_run, i_run = carry
            w = w_ref[pl.ds(k0 + kc * KT, KT), :]
            wsq = jnp.sum(w * w, axis=1)
            xw2 = lax.dot_general(zm2, w, (((1,), (1,)), ((), ())),
                                  preferred_element_type=jnp.float32)
            d = (xsq + wsq[None, :]) + xw2
            m = jnp.min(d, axis=1)
            kk = lax.broadcasted_iota(jnp.int32, (NT, KT), 1)
            li = jnp.min(jnp.where(d == m[:, None], kk, jnp.int32(2**30)),
                         axis=1) + (k0 + kc * KT)
            better = m < m_run
            return jnp.where(better, m, m_run), jnp.where(better, li, i_run)

        m0 = jnp.full((NT,), jnp.inf, jnp.float32)
        i0 = jnp.zeros((NT,), jnp.int32)
        return lax.fori_loop(0, (CODEBOOK // 2) // KT, body, (m0, i0))

    m1, i1 = half_argmin(0)
    m2, i2 = half_argmin(CODEBOOK // 2)
    m1b = m1.astype(jnp.bfloat16).astype(jnp.float32)
    take2 = m2 < m1b
    idx_ref[0, 0, :] = jnp.where(take2, i2, i1)
    mind_ref[0, 0, :] = jnp.where(take2, m2, m1)


def _dist_argmin(z_flat, weight):
    grid = (NTOK // NT,)
    return pl.pallas_call(
        _dist_argmin_kernel,
        grid=grid,
        in_specs=[
            pl.BlockSpec((NT, DIM), lambda i: (i, 0)),
            pl.BlockSpec((CODEBOOK, DIM), lambda i: (0, 0)),
        ],
        out_specs=(
            pl.BlockSpec((1, 1, NT), lambda i: (i, 0, 0)),
            pl.BlockSpec((1, 1, NT), lambda i: (i, 0, 0)),
        ),
        out_shape=(
            jax.ShapeDtypeStruct((NTOK // NT, 1, NT), jnp.int32),
            jax.ShapeDtypeStruct((NTOK // NT, 1, NT), jnp.float32),
        ),
    )(z_flat, weight)


def _sc_gather(weight, idx):
    w_pad = jnp.concatenate([weight, jnp.zeros_like(weight)], axis=1)
    idx2 = idx.reshape(1, NTOK)
    mesh = plsc.VectorSubcoreMesh(core_axis_name="core",
                                  subcore_axis_name="subcore")

    @pl.kernel(out_type=jax.ShapeDtypeStruct((NTOK, 2 * DIM), weight.dtype),
               mesh=mesh)
    def kern(w_hbm, i_hbm, o_hbm):
        def body(i_vmem, o_vmem):
            pltpu.sync_copy(w_hbm.at[i_vmem.at[0]], o_vmem)

        pltpu.emit_pipeline(
            body,
            grid=(NTOK // GW,),
            in_specs=[pl.BlockSpec((1, GW), lambda i: (0, i))],
            out_specs=[pl.BlockSpec((GW, 2 * DIM), lambda i: (i, 0))],
            core_axis_name="subcore",
            dimension_semantics=(pltpu.PARALLEL,),
        )(i_hbm, o_hbm)

    return kern(w_pad, idx2)[:, :DIM]


def kernel(z, weight):
    b, c, h, w = z.shape
    z_perm = jnp.transpose(z, (0, 2, 3, 1))
    nrm = jnp.linalg.norm(z_perm, axis=-1, keepdims=True)
    z_flat = (z_perm / jnp.clip(nrm, 1e-12, None)).reshape(NTOK, DIM)
    idx3, mind3 = _dist_argmin(z_flat, weight)
    idx = idx3.reshape(NTOK)
    loss = BETA * jnp.sum(mind3) / (NTOK * DIM)
    z_q = _sc_gather(weight, idx)
    z_q_out = jnp.transpose(z_q.reshape(b, h, w, c), (0, 3, 1, 2))
    return z_q_out, loss, idx


# speedup vs baseline: 3.8248x; 1.0078x over previous
import jax
import jax.numpy as jnp
from jax import lax
from jax.experimental import pallas as pl
from jax.experimental.pallas import tpu as pltpu
from jax.experimental.pallas import tpu_sc as plsc

CODEBOOK = 8192
DIM = 64
BETA = 0.25
NTOK = 8192

NT = 2048
KT = 4096
GW = 128


def _dist_argmin_kernel(z_ref, w_ref, idx_ref, mind_ref):
    zn = z_ref[...]
    xsq = jnp.sum(zn * zn, axis=1, keepdims=True)
    zm2 = -2.0 * zn

    def half_argmin(k0):
        def body(kc, carry):
            m_run, i_run = carry
            w = w_ref[pl.ds(k0 + kc * KT, KT), :]
            wsq = jnp.sum(w * w, axis=1)
            xw2 = lax.dot_general(zm2, w, (((1,), (1,)), ((), ())),
                                  preferred_element_type=jnp.float32)
            d = (xsq + wsq[None, :]) + xw2
            m = jnp.min(d, axis=1)
            kk = lax.broadcasted_iota(jnp.int32, (NT, KT), 1)
            li = jnp.min(jnp.where(d == m[:, None], kk, jnp.int32(2**30)),
                         axis=1) + (k0 + kc * KT)
            better = m < m_run
            return jnp.where(better, m, m_run), jnp.where(better, li, i_run)

        m0 = jnp.full((NT,), jnp.inf, jnp.float32)
        i0 = jnp.zeros((NT,), jnp.int32)
        return lax.fori_loop(0, (CODEBOOK // 2) // KT, body, (m0, i0))

    m1, i1 = half_argmin(0)
    m2, i2 = half_argmin(CODEBOOK // 2)
    m1b = m1.astype(jnp.bfloat16).astype(jnp.float32)
    take2 = m2 < m1b
    idx_ref[0, 0, :] = jnp.where(take2, i2, i1)
    mind_ref[0, 0, :] = jnp.where(take2, m2, m1)


def _dist_argmin(z_flat, weight):
    grid = (NTOK // NT,)
    return pl.pallas_call(
        _dist_argmin_kernel,
        grid=grid,
        in_specs=[
            pl.BlockSpec((NT, DIM), lambda i: (i, 0)),
            pl.BlockSpec((CODEBOOK, DIM), lambda i: (0, 0)),
        ],
        out_specs=(
            pl.BlockSpec((1, 1, NT), lambda i: (i, 0, 0)),
            pl.BlockSpec((1, 1, NT), lambda i: (i, 0, 0)),
        ),
        out_shape=(
            jax.ShapeDtypeStruct((NTOK // NT, 1, NT), jnp.int32),
            jax.ShapeDtypeStruct((NTOK // NT, 1, NT), jnp.float32),
        ),
    )(z_flat, weight)


def _sc_gather(weight, idx):
    w_pad = jnp.concatenate([weight, jnp.zeros_like(weight)], axis=1)
    idx2 = idx.reshape(1, NTOK)
    mesh = plsc.VectorSubcoreMesh(core_axis_name="core",
                                  subcore_axis_name="subcore")

    @pl.kernel(out_type=jax.ShapeDtypeStruct((NTOK, 2 * DIM), weight.dtype),
               mesh=mesh)
    def kern(w_hbm, i_hbm, o_hbm):
        def body(i_vmem, o_vmem):
            pltpu.sync_copy(w_hbm.at[i_vmem.at[0]], o_vmem)

        pltpu.emit_pipeline(
            body,
            grid=(NTOK // GW,),
            in_specs=[pl.BlockSpec((1, GW), lambda i: (0, i))],
            out_specs=[pl.BlockSpec((GW, 2 * DIM), lambda i: (i, 0))],
            core_axis_name="subcore",
            dimension_semantics=(pltpu.PARALLEL,),
        )(i_hbm, o_hbm)

    return kern(w_pad, idx2)[:, :DIM]


def kernel(z, weight):
    b, c, h, w = z.shape
    z_perm = jnp.transpose(z, (0, 2, 3, 1))
    nrm = jnp.linalg.norm(z_perm, axis=-1, keepdims=True)
    z_flat = (z_perm / jnp.clip(nrm, 1e-12, None)).reshape(NTOK, DIM)
    idx3, mind3 = _dist_argmin(z_flat, weight)
    idx = idx3.reshape(NTOK)
    loss = BETA * jnp.sum(mind3) / (NTOK * DIM)
    z_q = _sc_gather(weight, idx)
    z_q_out = jnp.transpose(z_q.reshape(b, h, w, c), (0, 3, 1, 2))
    return z_q_out, loss, idx
